# baseline (speedup 1.0000x reference)
import jax
import jax.numpy as jnp
from jax.experimental import pallas as pl
from jax.experimental.pallas import tpu as pltpu

B, C, N, R, NE = 16, 2048, 4096, 5, 4
C_BLK = 1024
C_STEPS = C // C_BLK


def _chowder_kernel(x_ref, add_ref, w_ref, cb_ref, W1_ref, b1_ref, W2_ref,
                    b2_ref, Wo_ref, bo_ref, Wl1_ref, bl1_ref, Wl2_ref,
                    bl2_ref, out_ref, acc_ref, agg_ref):
    b = pl.program_id(0)
    c = pl.program_id(1)
    part = jnp.sum(x_ref[0] * w_ref[...], axis=0, keepdims=True)

    @pl.when(c == 0)
    def _init():
        acc_ref[...] = part + cb_ref[0, 0]

    @pl.when(c > 0)
    def _acc():
        acc_ref[...] += part

    @pl.when(c == C_STEPS - 1)
    def _park():
        agg_ref[pl.ds(b, 1), :] = acc_ref[...]

    @pl.when((c == C_STEPS - 1) & (b == B - 1))
    def _tail():
        agg = agg_ref[...]
        iota = jax.lax.broadcasted_iota(jnp.int32, (B, N), 1)

        def extract(vals, largest, k):
            out = []
            cur = vals
            fill = -jnp.inf if largest else jnp.inf
            for _ in range(k):
                m = (jnp.max(cur, axis=1, keepdims=True) if largest
                     else jnp.min(cur, axis=1, keepdims=True))
                out.append(m)
                idx = jnp.min(jnp.where(cur == m, iota, N), axis=1,
                              keepdims=True)
                cur = jnp.where(iota == idx, fill, cur)
            return jnp.concatenate(out, axis=1)

        top5 = extract(agg, True, R)
        bot5 = extract(agg, False, R)

        af = add_ref[:, 0, :]
        feats = jnp.dot(af, Wl1_ref[...], preferred_element_type=jnp.float32)
        prob = jax.nn.sigmoid(feats + bl1_ref[...])
        fl = jnp.dot(prob, Wl2_ref[...],
                     preferred_element_type=jnp.float32) + bl2_ref[...]

        mil = jnp.concatenate([top5, bot5, fl], axis=1)
        h1 = jax.nn.sigmoid(
            jnp.dot(mil, W1_ref[...], preferred_element_type=jnp.float32)
            + b1_ref[...])
        h2 = jax.nn.sigmoid(
            jnp.dot(h1, W2_ref[...], preferred_element_type=jnp.float32)
            + b2_ref[...])
        o = jax.nn.sigmoid(
            jnp.dot(h2, Wo_ref[...], preferred_element_type=jnp.float32)
            + bo_ref[...])
        out_ref[...] = o.reshape(B, 1, 1)


@jax.jit
def _run(in_features, add_features, conv_w, conv_b, W1, b1, W2, b2, Wo, bo,
         Wl1, bl1, Wl2, bl2):
    w2d = conv_w.reshape(C, 1)
    cb = conv_b.reshape(1, 1)
    grid = (B, C_STEPS)
    const = lambda *shape: pl.BlockSpec(shape, lambda b, c: (0,) * len(shape))
    return pl.pallas_call(
        _chowder_kernel,
        grid=grid,
        in_specs=[
            pl.BlockSpec((1, C_BLK, N), lambda b, c: (b, c, 0)),
            const(B, 1, 3),
            pl.BlockSpec((C_BLK, 1), lambda b, c: (c, 0)),
            const(1, 1),
            const(2 * R + NE, 200),
            const(1, 200),
            const(200, 100),
            const(1, 100),
            const(100, 1),
            const(1, 1),
            const(3, 4),
            const(1, 4),
            const(4, NE),
            const(1, NE),
        ],
        out_specs=const(B, 1, 1),
        out_shape=jax.ShapeDtypeStruct((B, 1, 1), jnp.float32),
        scratch_shapes=[pltpu.VMEM((1, N), jnp.float32),
                        pltpu.VMEM((B, N), jnp.float32)],
    )(in_features, add_features.reshape(B, 1, 3), w2d, cb, W1,
      b1.reshape(1, 200), W2, b2.reshape(1, 100), Wo, bo.reshape(1, 1),
      Wl1, bl1.reshape(1, 4), Wl2, bl2.reshape(1, NE))


def kernel(in_features, add_features, conv_w, conv_b, W1, b1, W2, b2, Wo, bo,
           Wl1, bl1, Wl2, bl2):
    return _run(in_features, add_features, conv_w, conv_b, W1, b1, W2, b2,
                Wo, bo, Wl1, bl1, Wl2, bl2)


# speedup vs baseline: 1.0173x; 1.0173x over previous
import functools

import jax
import jax.numpy as jnp
from jax.experimental import pallas as pl
from jax.experimental.pallas import tpu as pltpu

B, C, N, R, NE = 16, 2048, 4096, 5, 4
C_BLK = 1024
C_STEPS = C // C_BLK


def _chowder_kernel(x_ref, add_ref, w_ref, cb_ref, W1_ref, b1_ref, W2_ref,
                    b2_ref, Wo_ref, bo_ref, Wl1_ref, bl1_ref, Wl2_ref,
                    bl2_ref, out_ref, acc_ref):
    c = pl.program_id(1)
    x = x_ref[0]
    w = w_ref[...]
    part = jnp.sum(x * w, axis=0, keepdims=True)

    @pl.when(c == 0)
    def _init():
        acc_ref[...] = part + cb_ref[0, 0]

    @pl.when(c > 0)
    def _acc():
        acc_ref[...] += part

    @pl.when(c == C_STEPS - 1)
    def _tail():
        agg = acc_ref[...]
        iota = jax.lax.broadcasted_iota(jnp.int32, (1, N), 1)

        def extract(vals, largest, k):
            out = []
            cur = vals
            fill = -jnp.inf if largest else jnp.inf
            for _ in range(k):
                m = (jnp.max(cur, axis=1, keepdims=True) if largest
                     else jnp.min(cur, axis=1, keepdims=True))
                out.append(m)
                idx = jnp.min(jnp.where(cur == m, iota, N), axis=1,
                              keepdims=True)
                cur = jnp.where(iota == idx, fill, cur)
            return jnp.concatenate(out, axis=1)

        top5 = extract(agg, True, R)
        bot5 = extract(agg, False, R)

        af = add_ref[0]
        feats = jnp.dot(af, Wl1_ref[...], preferred_element_type=jnp.float32)
        prob = jax.nn.sigmoid(feats + bl1_ref[...])
        fl = jnp.dot(prob, Wl2_ref[...],
                     preferred_element_type=jnp.float32) + bl2_ref[...]

        mil = jnp.concatenate([top5, bot5, fl], axis=1)
        h1 = jax.nn.sigmoid(
            jnp.dot(mil, W1_ref[...], preferred_element_type=jnp.float32)
            + b1_ref[...])
        h2 = jax.nn.sigmoid(
            jnp.dot(h1, W2_ref[...], preferred_element_type=jnp.float32)
            + b2_ref[...])
        o = jax.nn.sigmoid(
            jnp.dot(h2, Wo_ref[...], preferred_element_type=jnp.float32)
            + bo_ref[...])
        out_ref[...] = o.reshape(1, 1, 1)


@jax.jit
def _run(in_features, add_features, conv_w, conv_b, W1, b1, W2, b2, Wo, bo,
         Wl1, bl1, Wl2, bl2):
    w2d = conv_w.reshape(C, 1)
    cb = conv_b.reshape(1, 1)
    grid = (B, C_STEPS)
    const = lambda *shape: pl.BlockSpec(shape, lambda b, n: (0,) * len(shape))
    return pl.pallas_call(
        _chowder_kernel,
        grid=grid,
        in_specs=[
            pl.BlockSpec((1, C_BLK, N), lambda b, c: (b, c, 0)),
            pl.BlockSpec((1, 1, 3), lambda b, n: (b, 0, 0)),
            pl.BlockSpec((C_BLK, 1), lambda b, c: (c, 0)),
            const(1, 1),
            const(2 * R + NE, 200),
            const(1, 200),
            const(200, 100),
            const(1, 100),
            const(100, 1),
            const(1, 1),
            const(3, 4),
            const(1, 4),
            const(4, NE),
            const(1, NE),
        ],
        out_specs=pl.BlockSpec((1, 1, 1), lambda b, n: (b, 0, 0)),
        out_shape=jax.ShapeDtypeStruct((B, 1, 1), jnp.float32),
        scratch_shapes=[pltpu.VMEM((1, N), jnp.float32)],
        compiler_params=pltpu.CompilerParams(vmem_limit_bytes=110 * 1024 * 1024),
    )(in_features, add_features.reshape(B, 1, 3), w2d, cb, W1, b1.reshape(1, 200), W2,
      b2.reshape(1, 100), Wo, bo.reshape(1, 1), Wl1, bl1.reshape(1, 4), Wl2,
      bl2.reshape(1, NE))


def kernel(in_features, add_features, conv_w, conv_b, W1, b1, W2, b2, Wo, bo,
           Wl1, bl1, Wl2, bl2):
    return _run(in_features, add_features, conv_w, conv_b, W1, b1, W2, b2,
                Wo, bo, Wl1, bl1, Wl2, bl2)
